# baseline (device time: 18272 ns/iter reference)
import jax
import jax.numpy as jnp
from jax import lax
from jax.experimental import pallas as pl
from jax.experimental.pallas import tpu as pltpu

N_Z = 4
B, H, D, BS = 8, 8, 64, 16
PAGES_PER_SHARD = 64
KEYS = PAGES_PER_SHARD * BS
NSLOT = 64
SCALE = D ** -0.5
NEG = -1e30


def kernel(Q, K, V, bt, lens):
    Q2 = Q.reshape(B, H * D)
    K2 = K.reshape(KEYS, H * D)
    V2 = V.reshape(KEYS, H * D)

    def body(q_ref, k_ref, v_ref, bt_ref, lens_ref, out_ref,
             comm_ref, w_ref, send_sems, recv_sems):
        my_x = lax.axis_index("x")
        my_y = lax.axis_index("y")
        my_z = lax.axis_index("z")

        z_off = my_z * PAGES_PER_SHARD
        pid_f = (z_off
                 + lax.broadcasted_iota(jnp.int32, (NSLOT, KEYS), 1) // BS
                 ).astype(jnp.float32)
        slot_iota = lax.broadcasted_iota(jnp.int32, (NSLOT, KEYS), 0)

        ii = lax.broadcasted_iota(jnp.int32, (NSLOT, NSLOT), 0)
        jj = lax.broadcasted_iota(jnp.int32, (NSLOT, NSLOT), 1)
        ident = jnp.where(ii == jj, 1.0, 0.0)
        btT = lax.dot_general(
            ident, bt_ref[:, :].astype(jnp.float32), (((1,), (1,)), ((), ())),
            preferred_element_type=jnp.float32)

        for b in range(B):
            bt_col = btT[:, b:b + 1]
            used = jnp.logical_and(bt_col == pid_f, slot_iota < lens_ref[b])
            w_ref[b:b + 1, :] = jnp.sum(
                jnp.where(used, 1.0, 0.0), axis=0, keepdims=True)

        w = w_ref[:, :]
        has = w > 0.0
        for h in range(H):
            q_h = q_ref[:, h * D:(h + 1) * D]
            k_h = k_ref[:, h * D:(h + 1) * D]
            v_h = v_ref[:, h * D:(h + 1) * D]
            s = lax.dot_general(
                q_h, k_h, (((1,), (1,)), ((), ())),
                preferred_element_type=jnp.float32) * SCALE
            s = jnp.where(has, s, NEG)
            m = jnp.max(s, axis=1, keepdims=True)
            e = jnp.exp(s - m) * w
            l = jnp.sum(e, axis=1, keepdims=True)
            o = lax.dot_general(
                e, v_h, (((1,), (0,)), ((), ())),
                preferred_element_type=jnp.float32)
            comm_ref[0, 0, :, h, :] = o
            comm_ref[0, 1, :, h, :] = jnp.broadcast_to(m, (B, D))
            comm_ref[0, 2, :, h, :] = jnp.broadcast_to(l, (B, D))

        barrier_sem = pltpu.get_barrier_semaphore()
        for d in (1, 2, 3):
            pl.semaphore_signal(
                barrier_sem, inc=1,
                device_id=(my_x, my_y, (my_z + d) % N_Z),
                device_id_type=pl.DeviceIdType.MESH,
            )
        pl.semaphore_wait(barrier_sem, 3)

        rdmas = []
        for d in (1, 2, 3):
            dst_slot = N_Z - d
            rdma = pltpu.make_async_remote_copy(
                src_ref=comm_ref.at[0],
                dst_ref=comm_ref.at[dst_slot],
                send_sem=send_sems.at[d - 1],
                recv_sem=recv_sems.at[dst_slot - 1],
                device_id=(my_x, my_y, (my_z + d) % N_Z),
                device_id_type=pl.DeviceIdType.MESH,
            )
            rdma.start()
            rdmas.append(rdma)
        for rdma in rdmas:
            rdma.wait_recv()
        for rdma in rdmas:
            rdma.wait_send()

        o_all = comm_ref[:, 0]
        m_all = comm_ref[:, 1]
        l_all = comm_ref[:, 2]
        m_max = jnp.max(m_all, axis=0)
        alpha = jnp.exp(m_all - m_max[None])
        l_tot = jnp.sum(l_all * alpha, axis=0)
        out_ref[:, 0, :, :] = jnp.sum(o_all * alpha, axis=0) / l_tot

    return pl.pallas_call(
        body,
        out_shape=jax.ShapeDtypeStruct((B, 1, H, D), jnp.float32),
        in_specs=[
            pl.BlockSpec(memory_space=pltpu.VMEM),
            pl.BlockSpec(memory_space=pltpu.VMEM),
            pl.BlockSpec(memory_space=pltpu.VMEM),
            pl.BlockSpec(memory_space=pltpu.VMEM),
            pl.BlockSpec(memory_space=pltpu.SMEM),
        ],
        out_specs=pl.BlockSpec(memory_space=pltpu.VMEM),
        scratch_shapes=[
            pltpu.VMEM((N_Z, 3, B, H, D), jnp.float32),
            pltpu.VMEM((B, KEYS), jnp.float32),
            pltpu.SemaphoreType.DMA((3,)),
            pltpu.SemaphoreType.DMA((3,)),
        ],
        compiler_params=pltpu.CompilerParams(collective_id=0),
    )(Q2, K2, V2, bt, lens)
